# baseline (speedup 1.0000x reference)
import functools

import jax
import jax.numpy as jnp
from jax.experimental import pallas as pl
from jax.experimental.pallas import tpu as pltpu

N = 10000
NFEAT = 128
NHID = 32
NCLASS = 16
BLK = 200
NBLK = N // BLK
LAST = NBLK - 1


def _fused(x_ref, emb_ref, fcw_ref, fcb_ref, w1_ref, b1_ref, w2_ref, b2_ref,
           adj_hbm, out_hbm, adjb_hbm, z1_ref, z2_ref):

    def _logsoftmax(o):
        m = jnp.max(o, axis=1, keepdims=True)
        return o - (jnp.log(jnp.sum(jnp.exp(o - m), axis=1, keepdims=True)) + m)

    def body(adjf_blk, adjb_in, out_blk, adjb_out):
        p = pl.program_id(0)
        i = pl.program_id(1)
        row = pl.multiple_of(i * BLK, BLK)

        @pl.when(jnp.logical_and(p == 0, i == 0))
        def _prelude():
            w1a = w1_ref[:NFEAT]
            w1b = w1_ref[NFEAT:]
            wc = jnp.dot(fcw_ref[:], w1b, preferred_element_type=jnp.float32)
            c0 = jnp.dot(fcb_ref[:], w1b, preferred_element_type=jnp.float32)
            z1_ref[:] = (
                jnp.dot(x_ref[:], w1a, preferred_element_type=jnp.float32)
                + jnp.dot(emb_ref[:], wc, preferred_element_type=jnp.float32)
                + c0
            ).astype(jnp.bfloat16)

        @pl.when(p == 0)
        def _pass1():
            adjb_out[:] = adjf_blk[:].astype(jnp.bfloat16)
            h = jax.lax.dot_general(
                adjf_blk[:], z1_ref[:], (((1,), (0,)), ((), ())),
                preferred_element_type=jnp.float32)
            h = jnp.maximum(h + b1_ref[:], 0.0)
            z2_ref[pl.ds(row, BLK), :] = jnp.dot(
                h, w2_ref[:], preferred_element_type=jnp.float32
            ).astype(jnp.bfloat16)

        @pl.when(jnp.logical_and(p == 1, i < LAST))
        def _pass2():
            o = jnp.dot(adjb_in[:], z2_ref[:],
                        preferred_element_type=jnp.float32)
            out_blk[:] = _logsoftmax(o + b2_ref[:])

        @pl.when(jnp.logical_and(p == 1, i == LAST))
        def _pass2_last():
            o = jax.lax.dot_general(
                adjf_blk[:], z2_ref[:], (((1,), (0,)), ((), ())),
                preferred_element_type=jnp.float32)
            out_blk[:] = _logsoftmax(o + b2_ref[:])

    pltpu.emit_pipeline(
        body,
        grid=(2, NBLK),
        in_specs=[
            pl.BlockSpec(
                (BLK, N),
                lambda p, i: (jnp.where(p == 0, i, LAST), 0),
                pipeline_mode=pl.Buffered(buffer_count=2, use_lookahead=True),
            ),
            pl.BlockSpec(
                (BLK, N),
                lambda p, i: (jnp.where(p == 1, i, 0), 0),
                pipeline_mode=pl.Buffered(buffer_count=3),
            ),
        ],
        out_specs=[
            pl.BlockSpec(
                (BLK, NCLASS),
                lambda p, i: (jnp.where(p == 1, i, 0), 0),
            ),
            pl.BlockSpec(
                (BLK, N),
                lambda p, i: (jnp.where(p == 0, i, LAST), 0),
            ),
        ],
    )(adj_hbm, adjb_hbm, out_hbm, adjb_hbm)


_VMEM = pl.BlockSpec(memory_space=pltpu.VMEM)
_HBM = pl.BlockSpec(memory_space=pl.ANY)


@functools.partial(jax.jit, static_argnames=())
def kernel(x, adj, emb_table, fc_W, fc_b, W1, b1, W2, b2):
    out, _ = pl.pallas_call(
        _fused,
        in_specs=[_VMEM] * 8 + [_HBM],
        out_specs=(_HBM, _HBM),
        out_shape=(
            jax.ShapeDtypeStruct((N, NCLASS), jnp.float32),
            jax.ShapeDtypeStruct((N, N), jnp.bfloat16),
        ),
        scratch_shapes=[
            pltpu.VMEM((N, NHID), jnp.bfloat16),
            pltpu.VMEM((N, NCLASS), jnp.bfloat16),
        ],
    )(x, emb_table, fc_W, fc_b.reshape(1, -1), W1, b1.reshape(1, -1),
      W2, b2.reshape(1, -1), adj)
    return out


# speedup vs baseline: 1.1164x; 1.1164x over previous
import functools

import jax
import jax.numpy as jnp
from jax.experimental import pallas as pl
from jax.experimental.pallas import tpu as pltpu

N = 10000
NFEAT = 128
NHID = 32
NCLASS = 16
BLK = 200
ADJ_BUFS = 3
NBLK = N // BLK


def _fused(x_hbm, emb_hbm, fcw_ref, fcb_ref, w1_ref, b1_ref, w2_ref, b2_ref,
           adj_hbm, out_hbm, z1_ref, z2_ref):

    def body(adj_blk, x_ref, emb_ref, out_blk):
        p = pl.program_id(0)
        i = pl.program_id(1)
        row = pl.multiple_of(i * BLK, BLK)

        @pl.when(jnp.logical_and(p == 0, i == 0))
        def _prelude():
            w1a = w1_ref[:NFEAT]
            w1b = w1_ref[NFEAT:]
            wc = jnp.dot(fcw_ref[:], w1b, preferred_element_type=jnp.float32)
            c0 = jnp.dot(fcb_ref[:], w1b, preferred_element_type=jnp.float32)
            z1_ref[:] = (
                jnp.dot(x_ref[:], w1a, preferred_element_type=jnp.float32)
                + jnp.dot(emb_ref[:], wc, preferred_element_type=jnp.float32)
                + c0
            )

        @pl.when(p == 0)
        def _pass1():
            h = jnp.dot(adj_blk[:], z1_ref[:],
                        preferred_element_type=jnp.float32)
            h = jnp.maximum(h + b1_ref[:], 0.0)
            z2_ref[pl.ds(row, BLK), :] = jnp.dot(
                h, w2_ref[:], preferred_element_type=jnp.float32)

        @pl.when(p == 1)
        def _pass2():
            o = jnp.dot(adj_blk[:], z2_ref[:],
                        preferred_element_type=jnp.float32)
            o = o + b2_ref[:]
            m = jnp.max(o, axis=1, keepdims=True)
            lse = jnp.log(jnp.sum(jnp.exp(o - m), axis=1, keepdims=True)) + m
            out_blk[:] = o - lse

    pltpu.emit_pipeline(
        body,
        grid=(2, NBLK),
        in_specs=[
            pl.BlockSpec(
                (BLK, N), lambda p, i: (i, 0),
                pipeline_mode=pl.Buffered(buffer_count=ADJ_BUFS,
                                          use_lookahead=True),
            ),
            pl.BlockSpec((N, NFEAT), lambda p, i: (0, 0),
                         pipeline_mode=pl.Buffered(buffer_count=1)),
            pl.BlockSpec((N, 64), lambda p, i: (0, 0),
                         pipeline_mode=pl.Buffered(buffer_count=1)),
        ],
        out_specs=[pl.BlockSpec(
            (BLK, NCLASS),
            lambda p, i: (jnp.where(p == 1, i, 0), 0),
        )],
    )(adj_hbm, x_hbm, emb_hbm, out_hbm)


_VMEM = pl.BlockSpec(memory_space=pltpu.VMEM)
_HBM = pl.BlockSpec(memory_space=pl.ANY)


@functools.partial(jax.jit, static_argnames=())
def kernel(x, adj, emb_table, fc_W, fc_b, W1, b1, W2, b2):
    return pl.pallas_call(
        _fused,
        in_specs=[_HBM, _HBM] + [_VMEM] * 6 + [_HBM],
        out_specs=_HBM,
        out_shape=jax.ShapeDtypeStruct((N, NCLASS), jnp.float32),
        scratch_shapes=[
            pltpu.VMEM((N, NHID), jnp.float32),
            pltpu.VMEM((N, NCLASS), jnp.float32),
        ],
    )(x, emb_table, fc_W, fc_b.reshape(1, -1), W1, b1.reshape(1, -1),
      W2, b2.reshape(1, -1), adj)


# speedup vs baseline: 1.1251x; 1.0078x over previous
import functools

import jax
import jax.numpy as jnp
from jax.experimental import pallas as pl
from jax.experimental.pallas import tpu as pltpu

N = 10000
NFEAT = 128
NHID = 32
NCLASS = 16
BLK = 200
ADJ_BUFS = 3
NBLK = N // BLK


def _fused(x_ref, emb_ref, fcw_ref, fcb_ref, w1_ref, b1_ref, w2_ref, b2_ref,
           adj_hbm, out_hbm, z1_ref, z2_ref):

    def body(adj_blk, out_blk):
        p = pl.program_id(0)
        i = pl.program_id(1)
        row = pl.multiple_of(i * BLK, BLK)

        @pl.when(jnp.logical_and(p == 0, i == 0))
        def _prelude():
            w1a = w1_ref[:NFEAT]
            w1b = w1_ref[NFEAT:]
            wc = jnp.dot(fcw_ref[:], w1b, preferred_element_type=jnp.float32)
            c0 = jnp.dot(fcb_ref[:], w1b, preferred_element_type=jnp.float32)
            z1_ref[:] = (
                jnp.dot(x_ref[:], w1a, preferred_element_type=jnp.float32)
                + jnp.dot(emb_ref[:], wc, preferred_element_type=jnp.float32)
                + c0
            )

        @pl.when(p == 0)
        def _pass1():
            h = jnp.dot(adj_blk[:], z1_ref[:],
                        preferred_element_type=jnp.float32)
            h = jnp.maximum(h + b1_ref[:], 0.0)
            z2_ref[pl.ds(row, BLK), :] = jnp.dot(
                h, w2_ref[:], preferred_element_type=jnp.float32)

        @pl.when(p == 1)
        def _pass2():
            o = jnp.dot(adj_blk[:], z2_ref[:],
                        preferred_element_type=jnp.float32)
            o = o + b2_ref[:]
            m = jnp.max(o, axis=1, keepdims=True)
            lse = jnp.log(jnp.sum(jnp.exp(o - m), axis=1, keepdims=True)) + m
            out_blk[:] = o - lse

    pltpu.emit_pipeline(
        body,
        grid=(2, NBLK),
        in_specs=[pl.BlockSpec(
            (BLK, N), lambda p, i: (i, 0),
            pipeline_mode=pl.Buffered(buffer_count=ADJ_BUFS,
                                      use_lookahead=True),
        )],
        out_specs=[pl.BlockSpec(
            (BLK, NCLASS),
            lambda p, i: (jnp.where(p == 1, i, 0), 0),
        )],
    )(adj_hbm, out_hbm)


_VMEM = pl.BlockSpec(memory_space=pltpu.VMEM)
_HBM = pl.BlockSpec(memory_space=pl.ANY)


@functools.partial(jax.jit, static_argnames=())
def kernel(x, adj, emb_table, fc_W, fc_b, W1, b1, W2, b2):
    return pl.pallas_call(
        _fused,
        in_specs=[_VMEM] * 8 + [_HBM],
        out_specs=_HBM,
        out_shape=jax.ShapeDtypeStruct((N, NCLASS), jnp.float32),
        scratch_shapes=[
            pltpu.VMEM((N, NHID), jnp.float32),
            pltpu.VMEM((N, NCLASS), jnp.float32),
        ],
    )(x, emb_table, fc_W, fc_b.reshape(1, -1), W1, b1.reshape(1, -1),
      W2, b2.reshape(1, -1), adj)
